# baseline (speedup 1.0000x reference)
import functools

import jax
import jax.numpy as jnp
from jax import lax
from jax.experimental import pallas as pl
from jax.experimental.pallas import tpu as pltpu
from jax.experimental.pallas import tpu_sc as plsc

F = 32
B_USER = 16384
B_ITEM = 4096
NC = 2
NS = 16
L = 16
NW = NC * NS
UPW = B_USER // NW
IPS = B_ITEM // NS
CH = 128
N_UCH = UPW // CH
N_ICH = IPS // CH


def _mf_body(uf, itf, uidx, iidx, out,
             uidx_v, urows_v, iidx_v, irows_v, ps_v, part_v, out_v,
             shared, usem, isem):
  cid = lax.axis_index("c")
  sid = lax.axis_index("s")
  wid = sid * NC + cid
  ubase = wid * UPW
  zero = jnp.zeros((L,), jnp.float32)

  ucopies = []
  for t in range(N_UCH):
    pltpu.sync_copy(uidx.at[pl.ds(ubase + t * CH, CH)], uidx_v.at[t])
    ucopies.append(
        pltpu.async_copy(uf.at[uidx_v.at[t]],
                         urows_v.at[pl.ds(t * CH, CH)], usem))

  ibase = sid * IPS
  icopies = []
  for t in range(N_ICH):
    pltpu.sync_copy(iidx.at[pl.ds(ibase + t * CH, CH)], iidx_v.at[t])
    icopies.append(
        pltpu.async_copy(itf.at[iidx_v.at[t]],
                         irows_v.at[pl.ds(t * CH, CH)], isem))
  for c in icopies:
    c.wait()

  @pl.loop(0, IPS, init_carry=(zero, zero), unroll=8)
  def _item_acc(i, carry):
    a0, a1 = carry
    return (a0 + irows_v[i, pl.ds(0, L)], a1 + irows_v[i, pl.ds(L, L)])
  a0, a1 = _item_acc
  part_v[pl.ds(0, L)] = a0
  part_v[pl.ds(L, L)] = a1

  pltpu.sync_copy(part_v, shared.at[sid])
  plsc.subcore_barrier()
  pltpu.sync_copy(shared, ps_v)

  @pl.loop(0, NS, init_carry=(zero, zero), unroll=True)
  def _part_acc(i, carry):
    a0, a1 = carry
    return (a0 + ps_v[i, pl.ds(0, L)], a1 + ps_v[i, pl.ds(L, L)])
  s0, s1 = _part_acc

  for c in ucopies:
    c.wait()

  s_sc = [s0[f] for f in range(L)] + [s1[f] for f in range(L)]
  lane = lax.iota(jnp.int32, L)
  col_ids = [jnp.full((L,), f, jnp.int32) for f in range(F)]

  @pl.loop(0, UPW // L)
  def _dot_block(b):
    rows = b * L + lane
    acc = zero
    for f in range(F):
      col = plsc.load_gather(urows_v, [rows, col_ids[f]])
      acc = acc + col * s_sc[f]
    out_v[pl.ds(b * L, L)] = acc

  pltpu.sync_copy(out_v, out.at[pl.ds(ubase, UPW)])


_mf_kernel = pl.kernel(
    _mf_body,
    out_type=jax.ShapeDtypeStruct((B_USER,), jnp.float32),
    mesh=plsc.VectorSubcoreMesh(core_axis_name="c", subcore_axis_name="s"),
    compiler_params=pltpu.CompilerParams(
        needs_layout_passes=False, use_tc_tiling_on_sc=False),
    scratch_types=[
        pltpu.VMEM((N_UCH, CH), jnp.int32),
        pltpu.VMEM((UPW, F), jnp.float32),
        pltpu.VMEM((N_ICH, CH), jnp.int32),
        pltpu.VMEM((IPS, F), jnp.float32),
        pltpu.VMEM((NS, F), jnp.float32),
        pltpu.VMEM((F,), jnp.float32),
        pltpu.VMEM((UPW,), jnp.float32),
        pltpu.VMEM_SHARED((NS, F), jnp.float32),
        pltpu.SemaphoreType.DMA,
        pltpu.SemaphoreType.DMA,
    ],
)


def kernel(user_factors, item_factors, user_indices, item_indices):
  return _mf_kernel(user_factors, item_factors,
                    user_indices.astype(jnp.int32),
                    item_indices.astype(jnp.int32))
